# baseline (speedup 1.0000x reference)
import functools

import jax
import jax.numpy as jnp
import numpy as np
from jax import lax
from jax.experimental import pallas as pl
from jax.experimental.pallas import tpu as pltpu
from jax.experimental.pallas import tpu_sc as plsc

H = 8
DK = 32
D = H * DK
VOCAB = 32000
DFF = 1024
N = 512
E = 65536

NC = 2
NS = 16
NW = NC * NS
EPW = E // NW
NBINS = N * N
TBINS = 3 * NBINS
SLICE = TBINS // NS
ROWS_PW = N // NW


def _histogram(c, s, wid, edge_lists, cnt_sh, srcbuf, dstbuf, idx_v,
               ones_v, sem):
    ebase = wid * EPW
    for t, (esrc, edst) in enumerate(edge_lists):
        pltpu.sync_copy(esrc.at[pl.ds(ebase, EPW)], srcbuf)
        pltpu.sync_copy(edst.at[pl.ds(ebase, EPW)], dstbuf)
        for j in range(16):
            for k in range(8):
                off = (j * 8 + k) * 16
                idx_v[j, pl.ds(k * 16, 16)] = (
                    dstbuf[pl.ds(off, 16)] * N
                    + srcbuf[pl.ds(off, 16)]
                    + t * NBINS)
        descs = [pltpu.async_copy(ones_v, cnt_sh.at[idx_v.at[j]], sem,
                                  add=True)
                 for j in range(16)]
        for d in descs:
            d.wait()


def _hist_out(c, s, ntypes, cnt_sh, cnt_out, sem):
    zrows = ntypes * N // NS
    odescs = [pltpu.async_copy(
        cnt_sh.at[pl.ds((s * zrows + r) * N, N)],
        cnt_out.at[c, s * zrows + r], sem)
        for r in range(zrows)]
    for dsc in odescs:
        dsc.wait()


def _sc1_body(ee_src, ee_dst,
              src_tok, tgt_tok, pos_tab,
              src_tokens, src_pos, tgt_tokens, tgt_pos,
              ones_in, zeros_in,
              cnt_out, xe_tok_out, xe_pos_out, xd_tok_out, xd_pos_out,
              srcbuf, dstbuf, idx_v, ones_v,
              tokidx, posidx, trows, prows, cnt_sh, sem, esem):
    c = lax.axis_index("c")
    s = lax.axis_index("s")
    wid = c * NS + s
    base = wid * ROWS_PW

    pltpu.sync_copy(ones_in, ones_v)
    pltpu.sync_copy(zeros_in, cnt_sh.at[pl.ds(s * (NBINS // NS), NBINS // NS)])
    plsc.subcore_barrier()

    ebase = wid * EPW
    pltpu.sync_copy(ee_src.at[pl.ds(ebase, EPW)], srcbuf)
    pltpu.sync_copy(ee_dst.at[pl.ds(ebase, EPW)], dstbuf)
    for j in range(16):
        for k in range(8):
            off = (j * 8 + k) * 16
            idx_v[j, pl.ds(k * 16, 16)] = (
                dstbuf[pl.ds(off, 16)] * N + srcbuf[pl.ds(off, 16)])
    descs = [pltpu.async_copy(ones_v, cnt_sh.at[idx_v.at[j]], sem, add=True)
             for j in range(16)]

    def _embed(tok_tab, tok_ids, pos_ids, tok_out, pos_out):
        pltpu.sync_copy(tok_ids.at[pl.ds(base, ROWS_PW)], tokidx)
        pltpu.sync_copy(pos_ids.at[pl.ds(base, ROWS_PW)], posidx)
        pltpu.async_copy(tok_tab.at[tokidx], trows, esem).wait()
        pltpu.async_copy(pos_tab.at[posidx], prows, esem).wait()
        pltpu.sync_copy(trows, tok_out.at[pl.ds(base, ROWS_PW)])
        pltpu.sync_copy(prows, pos_out.at[pl.ds(base, ROWS_PW)])

    _embed(src_tok, src_tokens, src_pos, xe_tok_out, xe_pos_out)
    _embed(tgt_tok, tgt_tokens, tgt_pos, xd_tok_out, xd_pos_out)

    for d in descs:
        d.wait()
    plsc.subcore_barrier()
    _hist_out(c, s, 1, cnt_sh, cnt_out, sem)


def _sc2_body(dd_src, dd_dst, ed_src, ed_dst,
              ones_in, zeros_in,
              cnt_out,
              srcbuf, dstbuf, idx_v, ones_v, cnt_sh, sem):
    c = lax.axis_index("c")
    s = lax.axis_index("s")
    wid = c * NS + s

    pltpu.sync_copy(ones_in, ones_v)
    pltpu.sync_copy(zeros_in,
                    cnt_sh.at[pl.ds(s * (2 * NBINS // NS), 2 * NBINS // NS)])
    plsc.subcore_barrier()
    _histogram(c, s, wid, ((dd_src, dd_dst), (ed_src, ed_dst)), cnt_sh,
               srcbuf, dstbuf, idx_v, ones_v, sem)
    plsc.subcore_barrier()
    _hist_out(c, s, 2, cnt_sh, cnt_out, sem)


_EDGE_SCRATCH = (
    pltpu.VMEM((EPW,), jnp.int32),
    pltpu.VMEM((EPW,), jnp.int32),
    pltpu.VMEM((16, 128), jnp.int32),
    pltpu.VMEM((128,), jnp.float32),
)


def _sc_prep(ee_src, ee_dst, dd_src, dd_dst, ed_src, ed_dst,
             src_tok, tgt_tok, pos_tab,
             src_tokens, src_pos, tgt_tokens, tgt_pos):
    mesh = plsc.VectorSubcoreMesh(core_axis_name="c", subcore_axis_name="s",
                                  num_cores=NC, num_subcores=NS)
    ones_in = jnp.ones((128,), jnp.float32)

    f1 = pl.kernel(
        _sc1_body,
        out_type=(
            jax.ShapeDtypeStruct((NC, N, N), jnp.float32),
            jax.ShapeDtypeStruct((N, D), jnp.float32),
            jax.ShapeDtypeStruct((N, D), jnp.float32),
            jax.ShapeDtypeStruct((N, D), jnp.float32),
            jax.ShapeDtypeStruct((N, D), jnp.float32),
        ),
        mesh=mesh,
        scratch_types=_EDGE_SCRATCH + (
            pltpu.VMEM((ROWS_PW,), jnp.int32),
            pltpu.VMEM((ROWS_PW,), jnp.int32),
            pltpu.VMEM((ROWS_PW, D), jnp.float32),
            pltpu.VMEM((ROWS_PW, D), jnp.float32),
            pltpu.VMEM_SHARED((NBINS,), jnp.float32),
            pltpu.SemaphoreType.DMA,
            pltpu.SemaphoreType.DMA,
        ),
    )
    cnt1, xe_tok, xe_pos, xd_tok, xd_pos = f1(
        ee_src, ee_dst, src_tok, tgt_tok, pos_tab,
        src_tokens, src_pos, tgt_tokens, tgt_pos,
        ones_in, jnp.zeros((NBINS // NS,), jnp.float32))

    f2 = pl.kernel(
        _sc2_body,
        out_type=jax.ShapeDtypeStruct((NC, 2 * N, N), jnp.float32),
        mesh=mesh,
        scratch_types=_EDGE_SCRATCH + (
            pltpu.VMEM_SHARED((2 * NBINS,), jnp.float32),
            pltpu.SemaphoreType.DMA,
        ),
    )
    cnt2 = f2(dd_src, dd_dst, ed_src, ed_dst,
              ones_in, jnp.zeros((2 * NBINS // NS,), jnp.float32))
    return cnt1, cnt2, xe_tok, xe_pos, xd_tok, xd_pos


def _mm(a, b):
    return jnp.dot(a.astype(jnp.bfloat16), b.astype(jnp.bfloat16),
                   preferred_element_type=jnp.float32)


def _layernorm(x, g, b):
    m = jnp.mean(x, axis=-1, keepdims=True)
    v = jnp.mean((x - m) ** 2, axis=-1, keepdims=True)
    return (x - m) / jnp.sqrt(v + 1e-5) * g + b


def _attn(xq, xkv, C, Wq, Wk, Wv, Wo):
    inv = np.float32(1.0 / np.sqrt(DK))
    q = (_mm(xq, Wq) * inv).astype(jnp.bfloat16)
    k = _mm(xkv, Wk).astype(jnp.bfloat16)
    v = _mm(xkv, Wv)
    outs = []
    for h in range(H):
        qh = q[:, h * DK:(h + 1) * DK]
        kh = k[:, h * DK:(h + 1) * DK]
        vh = v[:, h * DK:(h + 1) * DK]
        S = lax.dot_general(qh, kh, (((1,), (1,)), ((), ())),
                            preferred_element_type=jnp.float32)
        W = jnp.exp(jnp.clip(S, -10.0, 10.0)) * C
        wv = _mm(W, vh)
        z = jnp.sum(W, axis=1, keepdims=True)
        outs.append(wv / (z + 1e-9))
    o = jnp.concatenate(outs, axis=1)
    return _mm(o, Wo)


def _ffn(x, W1, b1, W2, b2):
    h = jax.nn.relu(_mm(x, W1) + b1)
    return _mm(h, W2) + b2


def _enc_kernel(enc_tree, *refs):
    xt_ref, xp_ref, cnt_ref = refs[0], refs[1], refs[2]
    out_ref = refs[-1]
    enc_params = jax.tree.unflatten(enc_tree, refs[3:-1])

    Cee = cnt_ref[0] + cnt_ref[1]
    x = xt_ref[...] + xp_ref[...]
    for p in enc_params:
        x = _layernorm(
            x + _attn(x, x, Cee, p['Wq'][...], p['Wk'][...], p['Wv'][...],
                      p['Wo'][...]),
            p['ln1_g'][...], p['ln1_b'][...])
        x = _layernorm(x + _ffn(x, p['W1'][...], p['b1'][...],
                                p['W2'][...], p['b2'][...]),
                       p['ln2_g'][...], p['ln2_b'][...])
    out_ref[...] = x


BVL = 3200
KVL = VOCAB // BVL


def _dec_kernel(dec_tree, *refs):
    (x_enc_ref, xt_ref, xp_ref, cnt_ref, wg_ref, bg_ref) = refs[0:6]
    xdec_out, logz_out = refs[-5], refs[-4]
    xb_sc, m_sc, s_sc = refs[-3], refs[-2], refs[-1]
    dec_params = jax.tree.unflatten(dec_tree, refs[6:-5])
    j = pl.program_id(0)

    @pl.when(j == 0)
    def _():
        Cdd = cnt_ref[0, 0 * N:1 * N] + cnt_ref[1, 0 * N:1 * N]
        Ced = cnt_ref[0, 1 * N:2 * N] + cnt_ref[1, 1 * N:2 * N]
        x_enc = x_enc_ref[...]
        x = xt_ref[...] + xp_ref[...]
        for p in dec_params:
            x = _layernorm(
                x + _attn(x, x, Cdd, p['Wq'][...], p['Wk'][...],
                          p['Wv'][...], p['Wo'][...]),
                p['ln1_g'][...], p['ln1_b'][...])
            x = _layernorm(
                x + _attn(x, x_enc, Ced, p['Wq2'][...], p['Wk2'][...],
                          p['Wv2'][...], p['Wo2'][...]),
                p['ln2_g'][...], p['ln2_b'][...])
            x = _layernorm(x + _ffn(x, p['W1'][...], p['b1'][...],
                                    p['W2'][...], p['b2'][...]),
                           p['ln3_g'][...], p['ln3_b'][...])
        xb = x.astype(jnp.bfloat16)
        xdec_out[...] = xb
        xb_sc[...] = xb

    l = jnp.dot(xb_sc[...], wg_ref[...],
                preferred_element_type=jnp.float32) + bg_ref[...]
    bm = jnp.max(l, axis=1, keepdims=True)

    @pl.when(j == 0)
    def _():
        m_sc[...] = bm
        s_sc[...] = jnp.sum(jnp.exp(l - bm), axis=1, keepdims=True)

    @pl.when(j > 0)
    def _():
        m_old = m_sc[...]
        m_new = jnp.maximum(m_old, bm)
        s_sc[...] = (s_sc[...] * jnp.exp(m_old - m_new)
                     + jnp.sum(jnp.exp(l - m_new), axis=1, keepdims=True))
        m_sc[...] = m_new

    @pl.when(j == KVL - 1)
    def _():
        logz_out[...] = m_sc[...] + jnp.log(s_sc[...])


_BODY_PARAMS = pltpu.CompilerParams(vmem_limit_bytes=100 * 1024 * 1024)


def _body(xe_tok, xe_pos, xd_tok, xd_pos, cnt1, cnt2, wgb, bg2,
          enc_params, dec_params):
    enc_leaves, enc_tree = jax.tree.flatten(enc_params)
    dec_leaves, dec_tree = jax.tree.flatten(dec_params)
    x_enc = pl.pallas_call(
        functools.partial(_enc_kernel, enc_tree),
        out_shape=jax.ShapeDtypeStruct((N, D), jnp.float32),
        compiler_params=_BODY_PARAMS,
    )(xe_tok, xe_pos, cnt1, *enc_leaves)

    def _const_spec(x):
        nd = len(x.shape)
        return pl.BlockSpec(x.shape, lambda j, _n=nd: (0,) * _n)

    return pl.pallas_call(
        functools.partial(_dec_kernel, dec_tree),
        grid=(KVL,),
        in_specs=[
            _const_spec(x_enc), _const_spec(xd_tok), _const_spec(xd_pos),
            _const_spec(cnt2),
            pl.BlockSpec((D, BVL), lambda j: (0, j)),
            pl.BlockSpec((1, BVL), lambda j: (0, j)),
        ] + [_const_spec(w) for w in dec_leaves],
        out_specs=[
            pl.BlockSpec((N, D), lambda j: (0, 0)),
            pl.BlockSpec((N, 1), lambda j: (0, 0)),
        ],
        out_shape=[
            jax.ShapeDtypeStruct((N, D), jnp.bfloat16),
            jax.ShapeDtypeStruct((N, 1), jnp.float32),
        ],
        scratch_shapes=[pltpu.VMEM((N, D), jnp.bfloat16),
                        pltpu.VMEM((N, 1), jnp.float32),
                        pltpu.VMEM((N, 1), jnp.float32)],
        compiler_params=_BODY_PARAMS,
    )(x_enc, xd_tok, xd_pos, cnt2, wgb, bg2, *dec_leaves)


BV = 6400
KV = VOCAB // BV


def _gen_out_kernel(x_ref, wg_ref, bg_ref, lz_ref, out_ref):
    l = jnp.dot(x_ref[...], wg_ref[...],
                preferred_element_type=jnp.float32) + bg_ref[...]
    out_ref[...] = l - lz_ref[...]


def _generator(xb, logz, wgb, bg2):
    return pl.pallas_call(
        _gen_out_kernel,
        grid=(KV,),
        in_specs=[
            pl.BlockSpec((N, D), lambda j: (0, 0)),
            pl.BlockSpec((D, BV), lambda j: (0, j)),
            pl.BlockSpec((1, BV), lambda j: (0, j)),
            pl.BlockSpec((N, 1), lambda j: (0, 0)),
        ],
        out_specs=pl.BlockSpec((N, BV), lambda j: (0, j)),
        out_shape=jax.ShapeDtypeStruct((N, VOCAB), jnp.float32),
    )(xb, wgb, bg2, logz)


def kernel(params, src_tokens, src_pos, tgt_tokens, tgt_pos,
           ee_src, ee_dst, dd_src, dd_dst, ed_src, ed_dst):
    cnt1, cnt2, xe_tok, xe_pos, xd_tok, xd_pos = _sc_prep(
        ee_src, ee_dst, dd_src, dd_dst, ed_src, ed_dst,
        params['src_tok'], params['tgt_tok'], params['pos'],
        src_tokens, src_pos, tgt_tokens, tgt_pos)
    wgb = params['Wg'].astype(jnp.bfloat16)
    bg2 = params['bg'].reshape(1, VOCAB)
    x_dec, logz = _body(xe_tok, xe_pos, xd_tok, xd_pos, cnt1, cnt2,
                        wgb, bg2, params['enc'], params['dec'])
    return _generator(x_dec, logz, wgb, bg2)


# speedup vs baseline: 1.0084x; 1.0084x over previous
import functools

import jax
import jax.numpy as jnp
import numpy as np
from jax import lax
from jax.experimental import pallas as pl
from jax.experimental.pallas import tpu as pltpu
from jax.experimental.pallas import tpu_sc as plsc

H = 8
DK = 32
D = H * DK
VOCAB = 32000
DFF = 1024
N = 512
E = 65536

NC = 2
NS = 16
NW = NC * NS
EPW = E // NW
NBINS = N * N
TBINS = 3 * NBINS
SLICE = TBINS // NS
ROWS_PW = N // NW


def _histogram(c, s, wid, edge_lists, cnt_sh, srcbuf, dstbuf, idx_v,
               ones_v, sem):
    ebase = wid * EPW
    for t, (esrc, edst) in enumerate(edge_lists):
        pltpu.sync_copy(esrc.at[pl.ds(ebase, EPW)], srcbuf)
        pltpu.sync_copy(edst.at[pl.ds(ebase, EPW)], dstbuf)
        for j in range(16):
            for k in range(8):
                off = (j * 8 + k) * 16
                idx_v[j, pl.ds(k * 16, 16)] = (
                    dstbuf[pl.ds(off, 16)] * N
                    + srcbuf[pl.ds(off, 16)]
                    + t * NBINS)
        descs = [pltpu.async_copy(ones_v, cnt_sh.at[idx_v.at[j]], sem,
                                  add=True)
                 for j in range(16)]
        for d in descs:
            d.wait()


def _hist_out(c, s, ntypes, cnt_sh, cnt_out, sem):
    zrows = ntypes * N // NS
    odescs = [pltpu.async_copy(
        cnt_sh.at[pl.ds((s * zrows + r) * N, N)],
        cnt_out.at[c, s * zrows + r], sem)
        for r in range(zrows)]
    for dsc in odescs:
        dsc.wait()


def _sc1_body(ee_src, ee_dst,
              src_tok, tgt_tok, pos_tab,
              src_tokens, src_pos, tgt_tokens, tgt_pos,
              ones_in, zeros_in,
              cnt_out, xe_tok_out, xe_pos_out, xd_tok_out, xd_pos_out,
              srcbuf, dstbuf, idx_v, ones_v,
              tokidx, posidx, trows, prows,
              tokidx2, posidx2, trows2, prows2, cnt_sh, sem, esem):
    c = lax.axis_index("c")
    s = lax.axis_index("s")
    wid = c * NS + s
    base = wid * ROWS_PW

    pltpu.sync_copy(ones_in, ones_v)
    pltpu.sync_copy(zeros_in, cnt_sh.at[pl.ds(s * (NBINS // NS), NBINS // NS)])
    plsc.subcore_barrier()

    ebase = wid * EPW
    pltpu.sync_copy(ee_src.at[pl.ds(ebase, EPW)], srcbuf)
    pltpu.sync_copy(ee_dst.at[pl.ds(ebase, EPW)], dstbuf)
    for j in range(16):
        for k in range(8):
            off = (j * 8 + k) * 16
            idx_v[j, pl.ds(k * 16, 16)] = (
                dstbuf[pl.ds(off, 16)] * N + srcbuf[pl.ds(off, 16)])
    descs = [pltpu.async_copy(ones_v, cnt_sh.at[idx_v.at[j]], sem, add=True)
             for j in range(16)]

    pltpu.sync_copy(src_tokens.at[pl.ds(base, ROWS_PW)], tokidx)
    pltpu.sync_copy(src_pos.at[pl.ds(base, ROWS_PW)], posidx)
    pltpu.sync_copy(tgt_tokens.at[pl.ds(base, ROWS_PW)], tokidx2)
    pltpu.sync_copy(tgt_pos.at[pl.ds(base, ROWS_PW)], posidx2)
    g1 = pltpu.async_copy(src_tok.at[tokidx], trows, esem)
    g2 = pltpu.async_copy(pos_tab.at[posidx], prows, esem)
    g3 = pltpu.async_copy(tgt_tok.at[tokidx2], trows2, esem)
    g4 = pltpu.async_copy(pos_tab.at[posidx2], prows2, esem)
    g1.wait()
    g2.wait()
    g3.wait()
    g4.wait()
    pltpu.sync_copy(trows, xe_tok_out.at[pl.ds(base, ROWS_PW)])
    pltpu.sync_copy(prows, xe_pos_out.at[pl.ds(base, ROWS_PW)])
    pltpu.sync_copy(trows2, xd_tok_out.at[pl.ds(base, ROWS_PW)])
    pltpu.sync_copy(prows2, xd_pos_out.at[pl.ds(base, ROWS_PW)])

    for d in descs:
        d.wait()
    plsc.subcore_barrier()
    _hist_out(c, s, 1, cnt_sh, cnt_out, sem)


def _sc2_body(dd_src, dd_dst, ed_src, ed_dst,
              ones_in, zeros_in,
              cnt_out,
              srcbuf, dstbuf, idx_v, ones_v, cnt_sh, sem):
    c = lax.axis_index("c")
    s = lax.axis_index("s")
    wid = c * NS + s

    pltpu.sync_copy(ones_in, ones_v)
    pltpu.sync_copy(zeros_in,
                    cnt_sh.at[pl.ds(s * (2 * NBINS // NS), 2 * NBINS // NS)])
    plsc.subcore_barrier()
    _histogram(c, s, wid, ((dd_src, dd_dst), (ed_src, ed_dst)), cnt_sh,
               srcbuf, dstbuf, idx_v, ones_v, sem)
    plsc.subcore_barrier()
    _hist_out(c, s, 2, cnt_sh, cnt_out, sem)


_EDGE_SCRATCH = (
    pltpu.VMEM((EPW,), jnp.int32),
    pltpu.VMEM((EPW,), jnp.int32),
    pltpu.VMEM((16, 128), jnp.int32),
    pltpu.VMEM((128,), jnp.float32),
)


def _sc_prep(ee_src, ee_dst, dd_src, dd_dst, ed_src, ed_dst,
             src_tok, tgt_tok, pos_tab,
             src_tokens, src_pos, tgt_tokens, tgt_pos):
    mesh = plsc.VectorSubcoreMesh(core_axis_name="c", subcore_axis_name="s",
                                  num_cores=NC, num_subcores=NS)
    ones_in = jnp.ones((128,), jnp.float32)

    f1 = pl.kernel(
        _sc1_body,
        out_type=(
            jax.ShapeDtypeStruct((NC, N, N), jnp.float32),
            jax.ShapeDtypeStruct((N, D), jnp.float32),
            jax.ShapeDtypeStruct((N, D), jnp.float32),
            jax.ShapeDtypeStruct((N, D), jnp.float32),
            jax.ShapeDtypeStruct((N, D), jnp.float32),
        ),
        mesh=mesh,
        scratch_types=_EDGE_SCRATCH + (
            pltpu.VMEM((ROWS_PW,), jnp.int32),
            pltpu.VMEM((ROWS_PW,), jnp.int32),
            pltpu.VMEM((ROWS_PW, D), jnp.float32),
            pltpu.VMEM((ROWS_PW, D), jnp.float32),
            pltpu.VMEM((ROWS_PW,), jnp.int32),
            pltpu.VMEM((ROWS_PW,), jnp.int32),
            pltpu.VMEM((ROWS_PW, D), jnp.float32),
            pltpu.VMEM((ROWS_PW, D), jnp.float32),
            pltpu.VMEM_SHARED((NBINS,), jnp.float32),
            pltpu.SemaphoreType.DMA,
            pltpu.SemaphoreType.DMA,
        ),
    )
    cnt1, xe_tok, xe_pos, xd_tok, xd_pos = f1(
        ee_src, ee_dst, src_tok, tgt_tok, pos_tab,
        src_tokens, src_pos, tgt_tokens, tgt_pos,
        ones_in, jnp.zeros((NBINS // NS,), jnp.float32))

    f2 = pl.kernel(
        _sc2_body,
        out_type=jax.ShapeDtypeStruct((NC, 2 * N, N), jnp.float32),
        mesh=mesh,
        scratch_types=_EDGE_SCRATCH + (
            pltpu.VMEM_SHARED((2 * NBINS,), jnp.float32),
            pltpu.SemaphoreType.DMA,
        ),
    )
    cnt2 = f2(dd_src, dd_dst, ed_src, ed_dst,
              ones_in, jnp.zeros((2 * NBINS // NS,), jnp.float32))
    return cnt1, cnt2, xe_tok, xe_pos, xd_tok, xd_pos


def _mm(a, b):
    return jnp.dot(a.astype(jnp.bfloat16), b.astype(jnp.bfloat16),
                   preferred_element_type=jnp.float32)


def _layernorm(x, g, b):
    m = jnp.mean(x, axis=-1, keepdims=True)
    v = jnp.mean((x - m) ** 2, axis=-1, keepdims=True)
    return (x - m) / jnp.sqrt(v + 1e-5) * g + b


def _attn(xq, xkv, C, Wq, Wk, Wv, Wo):
    inv = np.float32(1.0 / np.sqrt(DK))
    q = (_mm(xq, Wq) * inv).astype(jnp.bfloat16)
    k = _mm(xkv, Wk).astype(jnp.bfloat16)
    v = _mm(xkv, Wv)
    outs = []
    for h in range(H):
        qh = q[:, h * DK:(h + 1) * DK]
        kh = k[:, h * DK:(h + 1) * DK]
        vh = v[:, h * DK:(h + 1) * DK]
        S = lax.dot_general(qh, kh, (((1,), (1,)), ((), ())),
                            preferred_element_type=jnp.float32)
        W = jnp.exp(jnp.clip(S, -10.0, 10.0)) * C
        wv = _mm(W, vh)
        z = jnp.sum(W, axis=1, keepdims=True)
        outs.append(wv / (z + 1e-9))
    o = jnp.concatenate(outs, axis=1)
    return _mm(o, Wo)


def _ffn(x, W1, b1, W2, b2):
    h = jax.nn.relu(_mm(x, W1) + b1)
    return _mm(h, W2) + b2


def _enc_kernel(enc_tree, *refs):
    xt_ref, xp_ref, cnt_ref = refs[0], refs[1], refs[2]
    out_ref = refs[-1]
    enc_params = jax.tree.unflatten(enc_tree, refs[3:-1])

    Cee = cnt_ref[0] + cnt_ref[1]
    x = xt_ref[...] + xp_ref[...]
    for p in enc_params:
        x = _layernorm(
            x + _attn(x, x, Cee, p['Wq'][...], p['Wk'][...], p['Wv'][...],
                      p['Wo'][...]),
            p['ln1_g'][...], p['ln1_b'][...])
        x = _layernorm(x + _ffn(x, p['W1'][...], p['b1'][...],
                                p['W2'][...], p['b2'][...]),
                       p['ln2_g'][...], p['ln2_b'][...])
    out_ref[...] = x


BVL = 3200
KVL = VOCAB // BVL


def _dec_kernel(dec_tree, *refs):
    (x_enc_ref, xt_ref, xp_ref, cnt_ref, wg_ref, bg_ref) = refs[0:6]
    xdec_out, logz_out = refs[-5], refs[-4]
    xb_sc, m_sc, s_sc = refs[-3], refs[-2], refs[-1]
    dec_params = jax.tree.unflatten(dec_tree, refs[6:-5])
    j = pl.program_id(0)

    @pl.when(j == 0)
    def _():
        Cdd = cnt_ref[0, 0 * N:1 * N] + cnt_ref[1, 0 * N:1 * N]
        Ced = cnt_ref[0, 1 * N:2 * N] + cnt_ref[1, 1 * N:2 * N]
        x_enc = x_enc_ref[...]
        x = xt_ref[...] + xp_ref[...]
        for p in dec_params:
            x = _layernorm(
                x + _attn(x, x, Cdd, p['Wq'][...], p['Wk'][...],
                          p['Wv'][...], p['Wo'][...]),
                p['ln1_g'][...], p['ln1_b'][...])
            x = _layernorm(
                x + _attn(x, x_enc, Ced, p['Wq2'][...], p['Wk2'][...],
                          p['Wv2'][...], p['Wo2'][...]),
                p['ln2_g'][...], p['ln2_b'][...])
            x = _layernorm(x + _ffn(x, p['W1'][...], p['b1'][...],
                                    p['W2'][...], p['b2'][...]),
                           p['ln3_g'][...], p['ln3_b'][...])
        xb = x.astype(jnp.bfloat16)
        xdec_out[...] = xb
        xb_sc[...] = xb

    l = jnp.dot(xb_sc[...], wg_ref[...],
                preferred_element_type=jnp.float32) + bg_ref[...]
    bm = jnp.max(l, axis=1, keepdims=True)

    @pl.when(j == 0)
    def _():
        m_sc[...] = bm
        s_sc[...] = jnp.sum(jnp.exp(l - bm), axis=1, keepdims=True)

    @pl.when(j > 0)
    def _():
        m_old = m_sc[...]
        m_new = jnp.maximum(m_old, bm)
        s_sc[...] = (s_sc[...] * jnp.exp(m_old - m_new)
                     + jnp.sum(jnp.exp(l - m_new), axis=1, keepdims=True))
        m_sc[...] = m_new

    @pl.when(j == KVL - 1)
    def _():
        logz_out[...] = m_sc[...] + jnp.log(s_sc[...])


_BODY_PARAMS = pltpu.CompilerParams(vmem_limit_bytes=100 * 1024 * 1024)


def _body(xe_tok, xe_pos, xd_tok, xd_pos, cnt1, cnt2, wgb, bg2,
          enc_params, dec_params):
    enc_leaves, enc_tree = jax.tree.flatten(enc_params)
    dec_leaves, dec_tree = jax.tree.flatten(dec_params)
    x_enc = pl.pallas_call(
        functools.partial(_enc_kernel, enc_tree),
        out_shape=jax.ShapeDtypeStruct((N, D), jnp.float32),
        compiler_params=_BODY_PARAMS,
    )(xe_tok, xe_pos, cnt1, *enc_leaves)

    def _const_spec(x):
        nd = len(x.shape)
        return pl.BlockSpec(x.shape, lambda j, _n=nd: (0,) * _n)

    return pl.pallas_call(
        functools.partial(_dec_kernel, dec_tree),
        grid=(KVL,),
        in_specs=[
            _const_spec(x_enc), _const_spec(xd_tok), _const_spec(xd_pos),
            _const_spec(cnt2),
            pl.BlockSpec((D, BVL), lambda j: (0, j)),
            pl.BlockSpec((1, BVL), lambda j: (0, j)),
        ] + [_const_spec(w) for w in dec_leaves],
        out_specs=[
            pl.BlockSpec((N, D), lambda j: (0, 0)),
            pl.BlockSpec((N, 1), lambda j: (0, 0)),
        ],
        out_shape=[
            jax.ShapeDtypeStruct((N, D), jnp.bfloat16),
            jax.ShapeDtypeStruct((N, 1), jnp.float32),
        ],
        scratch_shapes=[pltpu.VMEM((N, D), jnp.bfloat16),
                        pltpu.VMEM((N, 1), jnp.float32),
                        pltpu.VMEM((N, 1), jnp.float32)],
        compiler_params=_BODY_PARAMS,
    )(x_enc, xd_tok, xd_pos, cnt2, wgb, bg2, *dec_leaves)


BV = 6400
KV = VOCAB // BV


def _gen_out_kernel(x_ref, wg_ref, bg_ref, lz_ref, out_ref):
    l = jnp.dot(x_ref[...], wg_ref[...],
                preferred_element_type=jnp.float32) + bg_ref[...]
    out_ref[...] = l - lz_ref[...]


def _generator(xb, logz, wgb, bg2):
    return pl.pallas_call(
        _gen_out_kernel,
        grid=(KV,),
        in_specs=[
            pl.BlockSpec((N, D), lambda j: (0, 0)),
            pl.BlockSpec((D, BV), lambda j: (0, j)),
            pl.BlockSpec((1, BV), lambda j: (0, j)),
            pl.BlockSpec((N, 1), lambda j: (0, 0)),
        ],
        out_specs=pl.BlockSpec((N, BV), lambda j: (0, j)),
        out_shape=jax.ShapeDtypeStruct((N, VOCAB), jnp.float32),
    )(xb, wgb, bg2, logz)


def kernel(params, src_tokens, src_pos, tgt_tokens, tgt_pos,
           ee_src, ee_dst, dd_src, dd_dst, ed_src, ed_dst):
    cnt1, cnt2, xe_tok, xe_pos, xd_tok, xd_pos = _sc_prep(
        ee_src, ee_dst, dd_src, dd_dst, ed_src, ed_dst,
        params['src_tok'], params['tgt_tok'], params['pos'],
        src_tokens, src_pos, tgt_tokens, tgt_pos)
    wgb = params['Wg'].astype(jnp.bfloat16)
    bg2 = params['bg'].reshape(1, VOCAB)
    x_dec, logz = _body(xe_tok, xe_pos, xd_tok, xd_pos, cnt1, cnt2,
                        wgb, bg2, params['enc'], params['dec'])
    return _generator(x_dec, logz, wgb, bg2)
